# baseline (speedup 1.0000x reference)
import jax
import jax.numpy as jnp
from jax.experimental import pallas as pl
from jax.experimental.pallas import tpu as pltpu

_LOCAL_SIZE = 5
_ALPHA = 1e-4
_BETA = 0.75


def _lrn_body(x_ref, o_ref):
    x = x_ref[0]
    sq = x * x
    _, h, w = sq.shape
    z1 = jnp.zeros((1, h, w), sq.dtype)
    z2 = jnp.zeros((2, h, w), sq.dtype)
    acc = sq
    acc = acc + jnp.concatenate([sq[1:], z1], axis=0)
    acc = acc + jnp.concatenate([sq[2:], z2], axis=0)
    acc = acc + jnp.concatenate([z1, sq[:-1]], axis=0)
    acc = acc + jnp.concatenate([z2, sq[:-2]], axis=0)
    t = 1.0 + (_ALPHA / _LOCAL_SIZE) * acc
    o_ref[0] = x * jnp.exp(-_BETA * jnp.log(t))


def kernel(x):
    n, c, h, w = x.shape
    return pl.pallas_call(
        _lrn_body,
        out_shape=jax.ShapeDtypeStruct(x.shape, x.dtype),
        grid=(n,),
        in_specs=[pl.BlockSpec((1, c, h, w), lambda i: (i, 0, 0, 0))],
        out_specs=pl.BlockSpec((1, c, h, w), lambda i: (i, 0, 0, 0)),
        compiler_params=pltpu.CompilerParams(
            dimension_semantics=("parallel",),
        ),
        name="lrn_fused",
    )(x)


# speedup vs baseline: 3.0350x; 3.0350x over previous
import jax
import jax.numpy as jnp
from jax.experimental import pallas as pl
from jax.experimental.pallas import tpu as pltpu

_LOCAL_SIZE = 5
_ALPHA = 1e-4
_BETA = 0.75


def _lrn_body(x_ref, o_ref):
    x = x_ref[0]
    sq = x * x
    h, w, c = sq.shape
    z1 = jnp.zeros((h, w, 1), sq.dtype)
    z2 = jnp.zeros((h, w, 2), sq.dtype)
    acc = sq
    acc = acc + jnp.concatenate([sq[..., 1:], z1], axis=-1)
    acc = acc + jnp.concatenate([sq[..., 2:], z2], axis=-1)
    acc = acc + jnp.concatenate([z1, sq[..., :-1]], axis=-1)
    acc = acc + jnp.concatenate([z2, sq[..., :-2]], axis=-1)
    t = 1.0 + (_ALPHA / _LOCAL_SIZE) * acc
    o_ref[0] = x * jnp.exp(-_BETA * jnp.log(t))


def kernel(x):
    n, c, h, w = x.shape
    xt = jnp.transpose(x, (0, 2, 3, 1))
    out = pl.pallas_call(
        _lrn_body,
        out_shape=jax.ShapeDtypeStruct((n, h, w, c), x.dtype),
        grid=(n,),
        in_specs=[pl.BlockSpec((1, h, w, c), lambda i: (i, 0, 0, 0))],
        out_specs=pl.BlockSpec((1, h, w, c), lambda i: (i, 0, 0, 0)),
        compiler_params=pltpu.CompilerParams(
            dimension_semantics=("parallel",),
        ),
        name="lrn_fused",
    )(xt)
    return jnp.transpose(out, (0, 3, 1, 2))


# speedup vs baseline: 5.6222x; 1.8525x over previous
import jax
import jax.numpy as jnp
from jax.experimental import pallas as pl
from jax.experimental.pallas import tpu as pltpu

_LOCAL_SIZE = 5
_ALPHA = 1e-4
_BETA = 0.75
_PAD = (_LOCAL_SIZE - 1) // 2


def _lrn_body(x_ref, b_ref, o_ref):
    x = x_ref[0]
    sq = (x * x).astype(jnp.bfloat16)
    acc = jax.lax.dot_general(
        sq, b_ref[...],
        (((1,), (0,)), ((), ())),
        preferred_element_type=jnp.float32,
    )
    t = 1.0 + acc
    o_ref[0] = x * jnp.exp2(-_BETA * jnp.log2(t))


def kernel(x):
    n, c, h, w = x.shape
    xt = jnp.transpose(x, (0, 2, 3, 1)).reshape(n, h * w, c)
    idx = jnp.arange(c)
    band = (jnp.abs(idx[:, None] - idx[None, :]) <= _PAD).astype(jnp.float32)
    bmat = (band * (_ALPHA / _LOCAL_SIZE)).astype(jnp.bfloat16)
    out = pl.pallas_call(
        _lrn_body,
        out_shape=jax.ShapeDtypeStruct((n, h * w, c), x.dtype),
        grid=(n,),
        in_specs=[
            pl.BlockSpec((1, h * w, c), lambda i: (i, 0, 0)),
            pl.BlockSpec((c, c), lambda i: (0, 0)),
        ],
        out_specs=pl.BlockSpec((1, h * w, c), lambda i: (i, 0, 0)),
        compiler_params=pltpu.CompilerParams(
            dimension_semantics=("parallel",),
        ),
        name="lrn_fused",
    )(xt, bmat)
    return jnp.transpose(out.reshape(n, h, w, c), (0, 3, 1, 2))


# speedup vs baseline: 6.1116x; 1.0870x over previous
import jax
import jax.numpy as jnp
from jax.experimental import pallas as pl
from jax.experimental.pallas import tpu as pltpu

_LOCAL_SIZE = 5
_ALPHA = 1e-4
_BETA = 0.75
_PAD = (_LOCAL_SIZE - 1) // 2


def _lrn_body(x_ref, b_ref, o_ref):
    for j in range(x_ref.shape[0]):
        x = x_ref[j]
        sq = (x * x).astype(jnp.bfloat16)
        acc = jax.lax.dot_general(
            sq, b_ref[...],
            (((1,), (0,)), ((), ())),
            preferred_element_type=jnp.float32,
        )
        t = 1.0 + acc
        o_ref[j] = x * jnp.exp2(-_BETA * jnp.log2(t))


def kernel(x):
    n, c, h, w = x.shape
    xt = jnp.transpose(x, (0, 2, 3, 1)).reshape(n, h * w, c)
    idx = jnp.arange(c)
    band = (jnp.abs(idx[:, None] - idx[None, :]) <= _PAD).astype(jnp.float32)
    bmat = (band * (_ALPHA / _LOCAL_SIZE)).astype(jnp.bfloat16)
    bn = 2
    out = pl.pallas_call(
        _lrn_body,
        out_shape=jax.ShapeDtypeStruct((n, h * w, c), x.dtype),
        grid=(n // bn,),
        in_specs=[
            pl.BlockSpec((bn, h * w, c), lambda i: (i, 0, 0)),
            pl.BlockSpec((c, c), lambda i: (0, 0)),
        ],
        out_specs=pl.BlockSpec((bn, h * w, c), lambda i: (i, 0, 0)),
        compiler_params=pltpu.CompilerParams(
            dimension_semantics=("parallel",),
        ),
        name="lrn_fused",
    )(xt, bmat)
    return jnp.transpose(out.reshape(n, h, w, c), (0, 3, 1, 2))


# speedup vs baseline: 6.1473x; 1.0058x over previous
import jax
import jax.numpy as jnp
from jax.experimental import pallas as pl
from jax.experimental.pallas import tpu as pltpu

_LOCAL_SIZE = 5
_ALPHA = 1e-4
_BETA = 0.75
_PAD = (_LOCAL_SIZE - 1) // 2


def _lrn_body(x_ref, b_ref, o_ref):
    for j in range(x_ref.shape[0]):
        x = x_ref[j]
        sq = (x * x).astype(jnp.bfloat16)
        acc = jax.lax.dot_general(
            sq, b_ref[...],
            (((1,), (0,)), ((), ())),
            preferred_element_type=jnp.float32,
        )
        t = 1.0 + acc
        o_ref[j] = x * jnp.exp2(-_BETA * jnp.log2(t))


def kernel(x):
    n, c, h, w = x.shape
    xt = jnp.transpose(x, (0, 2, 3, 1)).reshape(n, h * w, c)
    idx = jnp.arange(c)
    band = (jnp.abs(idx[:, None] - idx[None, :]) <= _PAD).astype(jnp.float32)
    bmat = (band * (_ALPHA / _LOCAL_SIZE)).astype(jnp.bfloat16)
    bn = 4
    out = pl.pallas_call(
        _lrn_body,
        out_shape=jax.ShapeDtypeStruct((n, h * w, c), x.dtype),
        grid=(n // bn,),
        in_specs=[
            pl.BlockSpec((bn, h * w, c), lambda i: (i, 0, 0)),
            pl.BlockSpec((c, c), lambda i: (0, 0)),
        ],
        out_specs=pl.BlockSpec((bn, h * w, c), lambda i: (i, 0, 0)),
        compiler_params=pltpu.CompilerParams(
            dimension_semantics=("parallel",),
            vmem_limit_bytes=56 * 1024 * 1024,
        ),
        name="lrn_fused",
    )(xt, bmat)
    return jnp.transpose(out.reshape(n, h, w, c), (0, 3, 1, 2))
